# baseline (speedup 1.0000x reference)
import functools

import jax
import jax.numpy as jnp
from jax import lax
from jax.experimental import pallas as pl
from jax.experimental.pallas import tpu as pltpu
from jax.experimental.pallas import tpu_sc as plsc

N = 10000
E = 320000
D = 128
EB = 128
NB = E // EB
NC = 2
NS = 16
NW = NC * NS
RB = 1000
HI = jax.lax.Precision.HIGHEST

_mesh = plsc.VectorSubcoreMesh(
    core_axis_name="c", subcore_axis_name="s", num_cores=NC, num_subcores=NS)


def _zero_2d(ref, nrows):
  def body(k, _):
    i = k // (D // 16)
    j = k % (D // 16)
    ref[i, pl.ds(j * 16, 16)] = jnp.zeros((16,), jnp.float32)
    return 0
  lax.fori_loop(0, nrows * (D // 16), body, 0)


def _fill_1d(ref, n, value):
  def body(k, _):
    ref[pl.ds(k * 16, 16)] = jnp.full((16,), value, jnp.float32)
    return 0
  lax.fori_loop(0, n // 16, body, 0)


@functools.partial(
    pl.kernel,
    out_type=(
        jax.ShapeDtypeStruct((NC, N, D), jnp.float32),
        jax.ShapeDtypeStruct((N,), jnp.float32),
        jax.ShapeDtypeStruct((N,), jnp.float32),
    ),
    mesh=_mesh,
    scratch_types=(
        pltpu.VMEM_SHARED((N, D), jnp.float32),
        pltpu.VMEM_SHARED((N,), jnp.float32),
        pltpu.VMEM((E // NW,), jnp.int32),
        [pltpu.VMEM((EB,), jnp.int32) for _ in range(4)],
        pltpu.VMEM((16,), jnp.int32),
        pltpu.VMEM((2, EB, D), jnp.float32),
        pltpu.VMEM((EB,), jnp.float32),
        pltpu.VMEM((1024,), jnp.float32),
        pltpu.SemaphoreType.DMA,
        pltpu.SemaphoreType.DMA,
        pltpu.SemaphoreType.DMA,
        pltpu.SemaphoreType.DMA,
    ),
)
def _sc1(z1, srcs, dsts, aggz_out, deg_out0, deg_out1,
         acc, degacc, src_v, dst_b, dst_t, rows_v, ones_v, zvec_v,
         sem_g, sem_s, sem_d, sem_i):
  c = lax.axis_index("c")
  s = lax.axis_index("s")
  gw = c * NS + s
  epw = E // NW
  nb = 78
  e_lo = gw * epw

  pltpu.async_copy(srcs.at[pl.ds(e_lo, epw)], src_v, sem_i)

  _zero_2d(rows_v.at[0], EB)
  _fill_1d(ones_v, EB, 1.0)
  _fill_1d(zvec_v, 1024, 0.0)

  def sidx(j):
    return src_v.at[pl.ds(j * EB, EB)]

  for j in range(6):
    pltpu.async_copy(rows_v.at[0].at[pl.ds(0, 104)],
                     acc.at[pl.ds(s * 624 + j * 104, 104)], sem_s)
  @pl.when(s == 0)
  def _():
    pltpu.sync_copy(rows_v.at[0].at[pl.ds(0, 16)], acc.at[pl.ds(NS * 624, 16)])
  @pl.when(s < 10)
  def _():
    pltpu.sync_copy(zvec_v.at[pl.ds(0, 1000)], degacc.at[pl.ds(s * 1000, 1000)])
  for j in range(6):
    pltpu.make_async_copy(rows_v.at[0].at[pl.ds(0, 104)],
                          acc.at[pl.ds(s * 624 + j * 104, 104)], sem_s).wait()
  pltpu.make_async_copy(srcs.at[pl.ds(e_lo, epw)], src_v, sem_i).wait()
  plsc.subcore_barrier()

  def idx_start(j, k):
    pltpu.async_copy(dsts.at[pl.ds(e_lo + j * EB, EB)], dst_b[k], sem_i)

  def idx_wait(k):
    pltpu.make_async_copy(dsts.at[pl.ds(0, EB)], dst_b[k], sem_i).wait()

  idx_start(0, 0)
  idx_start(1, 1)
  pltpu.async_copy(z1.at[sidx(0)], rows_v.at[0], sem_g)

  def outer(J, _):
    for k in range(4):
      j = J * 4 + k
      r = k % 2
      r1 = (k + 1) % 2
      kp = (k + 3) % 4
      k2 = (k + 2) % 4
      @pl.when(jnp.logical_and(j >= 1, j < nb + 1))
      def _():
        pltpu.make_async_copy(rows_v.at[r1], acc.at[dst_b[kp]], sem_s).wait()
      @pl.when(j + 1 < nb)
      def _():
        pltpu.async_copy(z1.at[sidx(j + 1)], rows_v.at[r1], sem_g)
      @pl.when(j < nb)
      def _():
        idx_wait(k)
        pltpu.make_async_copy(z1.at[sidx(j)], rows_v.at[r], sem_g).wait()
        pltpu.make_async_copy(rows_v.at[r], acc.at[dst_b[k]], sem_s
                              ).start(add=True)
        pltpu.make_async_copy(ones_v, degacc.at[dst_b[k]], sem_d
                              ).start(add=True)
      @pl.when(jnp.logical_and(j >= 1, j < nb + 1))
      def _():
        pltpu.make_async_copy(ones_v, degacc.at[dst_b[kp]], sem_d).wait()
      @pl.when(j + 2 < nb)
      def _():
        idx_start(j + 2, k2)
    return 0
  lax.fori_loop(0, (78 + 1 + 3) // 4 + 1, outer, 0)

  et = nb * EB
  rt = rows_v.at[0].at[pl.ds(0, 16)]
  pltpu.sync_copy(dsts.at[pl.ds(e_lo + et, 16)], dst_t)
  pltpu.async_copy(z1.at[src_v.at[pl.ds(et, 16)]], rt, sem_g).wait()
  pltpu.sync_copy(rt, acc.at[dst_t], add=True)
  pltpu.sync_copy(ones_v.at[pl.ds(0, 16)], degacc.at[dst_t], add=True)

  plsc.subcore_barrier()
  pltpu.sync_copy(acc.at[pl.ds(s * 624, 624)],
                  aggz_out.at[c, pl.ds(s * 624, 624)])
  @pl.when(s == 0)
  def _():
    pltpu.sync_copy(acc.at[pl.ds(NS * 624, 16)],
                    aggz_out.at[c, pl.ds(NS * 624, 16)])
  @pl.when(s < 10)
  def _():
    pltpu.sync_copy(degacc.at[pl.ds(s * 1000, 1000)], zvec_v.at[pl.ds(0, 1000)])
  @pl.when(jnp.logical_and(c == 0, s < 10))
  def _():
    pltpu.sync_copy(zvec_v.at[pl.ds(0, 1000)], deg_out0.at[pl.ds(s * 1000, 1000)])
  @pl.when(jnp.logical_and(c == 1, s < 10))
  def _():
    pltpu.sync_copy(zvec_v.at[pl.ds(0, 1000)], deg_out1.at[pl.ds(s * 1000, 1000)])


NH = N // NC
EPT = E // NS
NPT = 312


@functools.partial(
    pl.kernel,
    out_type=jax.ShapeDtypeStruct((N,), jnp.float32),
    mesh=_mesh,
    scratch_types=(
        pltpu.VMEM_SHARED((NH + 16,), jnp.float32),
        pltpu.VMEM((EPT,), jnp.int32),
        pltpu.VMEM((EPT,), jnp.int32),
        pltpu.VMEM((EPT,), jnp.float32),
        pltpu.VMEM((320,), jnp.float32),
        pltpu.VMEM((320,), jnp.float32),
        pltpu.VMEM((320,), jnp.float32),
        pltpu.VMEM((320,), jnp.float32),
        pltpu.VMEM((320,), jnp.float32),
        pltpu.VMEM((16,), jnp.float32),
        pltpu.VMEM((1024,), jnp.float32),
        pltpu.SemaphoreType.DMA,
    ),
)
def _sc2(n2, srcs, dsts, s2b, deg0, deg1, bias2, logits,
         acc, src_v, dstm_v, vals_v, av, s2v, dg0, dg1, ov, bv, zvec_v, sem):
  c = lax.axis_index("c")
  s = lax.axis_index("s")

  e0 = s * EPT
  pltpu.async_copy(srcs.at[pl.ds(e0, EPT)], src_v, sem)
  pltpu.async_copy(dsts.at[pl.ds(e0, EPT)], dstm_v, sem)

  _fill_1d(zvec_v, 1024, 0.0)
  @pl.when(s < 5)
  def _():
    pltpu.sync_copy(zvec_v.at[pl.ds(0, 1000)], acc.at[pl.ds(s * 1000, 1000)])
  @pl.when(s == 5)
  def _():
    pltpu.sync_copy(zvec_v.at[pl.ds(0, 16)], acc.at[pl.ds(NH, 16)])

  pltpu.make_async_copy(srcs.at[pl.ds(e0, EPT)], src_v, sem).wait()
  pltpu.make_async_copy(dsts.at[pl.ds(e0, EPT)], dstm_v, sem).wait()

  base = c * NH
  dump = NH + lax.iota(jnp.int32, 16)
  def remap(k, _):
    d = dstm_v[pl.ds(k * 16, 16)] - base
    oob = jnp.logical_or(d < 0, d >= NH)
    dstm_v[pl.ds(k * 16, 16)] = jnp.where(oob, dump, d)
    return 0
  lax.fori_loop(0, EPT // 16, remap, 0)

  plsc.subcore_barrier()
  pltpu.async_copy(n2.at[src_v], vals_v, sem).wait()
  pltpu.sync_copy(vals_v, acc.at[dstm_v], add=True)
  plsc.subcore_barrier()

  n0 = s * NPT
  g0 = base + n0
  pltpu.sync_copy(acc.at[pl.ds(n0, 320)], av)
  pltpu.sync_copy(s2b.at[pl.ds(g0, 320)], s2v)
  pltpu.sync_copy(deg0.at[pl.ds(g0, 320)], dg0)
  pltpu.sync_copy(deg1.at[pl.ds(g0, 320)], dg1)
  pltpu.sync_copy(bias2.at[pl.ds(0, 16)], bv)
  def fin(i, _):
    ix = pl.ds(i * 16, 16)
    deg = jnp.maximum(dg0[ix] + dg1[ix], 1.0)
    ov[ix] = s2v[ix] + bv[pl.ds(0, 16)] + av[ix] / deg
    return 0
  lax.fori_loop(0, 20, fin, 0)
  @pl.when(s < NS - 1)
  def _():
    pltpu.sync_copy(ov.at[pl.ds(0, NPT)], logits.at[pl.ds(g0, NPT)])
  @pl.when(s == NS - 1)
  def _():
    pltpu.sync_copy(ov, logits.at[pl.ds(g0, 320)])


def _tc1_body(x_ref, wn_ref, ws_ref, z1_ref, s1_ref):
  xb = x_ref[...]
  z1_ref[...] = jnp.dot(xb, wn_ref[...], preferred_element_type=jnp.float32,
                        precision=HI)
  s1_ref[...] = jnp.dot(xb, ws_ref[...], preferred_element_type=jnp.float32,
                        precision=HI)


def _tc2_body(s1_ref, aggz_ref, d0_ref, d1_ref, bs1_ref, bn1_ref,
              ws2_ref, wn2_ref, n2_ref, s2_ref):
  deg = jnp.maximum(d0_ref[...] + d1_ref[...], 1.0)
  inv = 1.0 / deg
  agg = (aggz_ref[0] + aggz_ref[1]) * inv
  h = jnp.maximum(s1_ref[...] + bs1_ref[...] + bn1_ref[...] + agg, 0.0)
  s2_ref[...] = jnp.dot(h, ws2_ref[...], preferred_element_type=jnp.float32,
                        precision=HI)
  n2_ref[...] = jnp.dot(h, wn2_ref[...], preferred_element_type=jnp.float32,
                        precision=HI)


def _tc1(x, wn, ws):
  return pl.pallas_call(
      _tc1_body,
      grid=(N // RB,),
      in_specs=[
          pl.BlockSpec((RB, D), lambda i: (i, 0)),
          pl.BlockSpec((D, D), lambda i: (0, 0)),
          pl.BlockSpec((D, D), lambda i: (0, 0)),
      ],
      out_specs=[
          pl.BlockSpec((RB, D), lambda i: (i, 0)),
          pl.BlockSpec((RB, D), lambda i: (i, 0)),
      ],
      out_shape=[
          jax.ShapeDtypeStruct((N, D), jnp.float32),
          jax.ShapeDtypeStruct((N, D), jnp.float32),
      ],
  )(x, wn, ws)


def _tc2(s1, aggz_p, d0, d1, bs1, bn1, ws2, wn2):
  return pl.pallas_call(
      _tc2_body,
      grid=(N // RB,),
      in_specs=[
          pl.BlockSpec((RB, D), lambda i: (i, 0)),
          pl.BlockSpec((NC, RB, D), lambda i: (0, i, 0)),
          pl.BlockSpec((RB, 1), lambda i: (i, 0)),
          pl.BlockSpec((RB, 1), lambda i: (i, 0)),
          pl.BlockSpec((1, D), lambda i: (0, 0)),
          pl.BlockSpec((1, D), lambda i: (0, 0)),
          pl.BlockSpec((D, 1), lambda i: (0, 0)),
          pl.BlockSpec((D, 1), lambda i: (0, 0)),
      ],
      out_specs=[
          pl.BlockSpec((RB, 1), lambda i: (i, 0)),
          pl.BlockSpec((RB, 1), lambda i: (i, 0)),
      ],
      out_shape=[
          jax.ShapeDtypeStruct((N, 1), jnp.float32),
          jax.ShapeDtypeStruct((N, 1), jnp.float32),
      ],
  )(s1, aggz_p, d0, d1, bs1, bn1, ws2, wn2)


def kernel(x, edge_index, W_self1, b_self1, W_neigh1, b_neigh1,
           W_self2, b_self2, W_neigh2, b_neigh2):
  src = edge_index[0].astype(jnp.int32)
  dst = edge_index[1].astype(jnp.int32)

  z1, s1 = _tc1(x, W_neigh1, W_self1)
  aggz_p, deg0, deg1 = _sc1(z1, src, dst)
  d0 = deg0.reshape(N, 1)
  d1 = deg1.reshape(N, 1)

  n2, s2 = _tc2(s1, aggz_p, d0, d1, b_self1.reshape(1, D),
                b_neigh1.reshape(1, D), W_self2, W_neigh2)

  bias2 = jnp.broadcast_to(b_self2 + b_neigh2, (16,))
  logits = _sc2(n2.reshape(N), src, dst, s2.reshape(N), deg0, deg1, bias2)
  return logits.reshape(N, 1)


# speedup vs baseline: 1.2708x; 1.2708x over previous
import functools

import jax
import jax.numpy as jnp
from jax import lax
from jax.experimental import pallas as pl
from jax.experimental.pallas import tpu as pltpu
from jax.experimental.pallas import tpu_sc as plsc

N = 10000
E = 320000
D = 128
EB = 128
NB = E // EB
NC = 2
NS = 16
NW = NC * NS
RB = 1000
HI = jax.lax.Precision.HIGHEST

_mesh = plsc.VectorSubcoreMesh(
    core_axis_name="c", subcore_axis_name="s", num_cores=NC, num_subcores=NS)


def _zero_2d(ref, nrows):
  def body(k, _):
    i = k // (D // 16)
    j = k % (D // 16)
    ref[i, pl.ds(j * 16, 16)] = jnp.zeros((16,), jnp.float32)
    return 0
  lax.fori_loop(0, nrows * (D // 16), body, 0)


def _fill_1d(ref, n, value):
  def body(k, _):
    ref[pl.ds(k * 16, 16)] = jnp.full((16,), value, jnp.float32)
    return 0
  lax.fori_loop(0, n // 16, body, 0)


@functools.partial(
    pl.kernel,
    out_type=(
        jax.ShapeDtypeStruct((NC, N, D), jnp.float32),
        jax.ShapeDtypeStruct((N,), jnp.float32),
        jax.ShapeDtypeStruct((N,), jnp.float32),
    ),
    mesh=_mesh,
    scratch_types=(
        pltpu.VMEM_SHARED((N, D), jnp.float32),
        pltpu.VMEM_SHARED((N,), jnp.float32),
        pltpu.VMEM((E // NW,), jnp.int32),
        [pltpu.VMEM((EB,), jnp.int32) for _ in range(4)],
        pltpu.VMEM((16,), jnp.int32),
        pltpu.VMEM((2, EB, D), jnp.float32),
        pltpu.VMEM((EB,), jnp.float32),
        pltpu.VMEM((1024,), jnp.float32),
        pltpu.SemaphoreType.DMA,
        pltpu.SemaphoreType.DMA,
        pltpu.SemaphoreType.DMA,
        pltpu.SemaphoreType.DMA,
    ),
)
def _sc1(z1, srcs, dsts, aggz_out, deg_out0, deg_out1,
         acc, degacc, src_v, dst_b, dst_t, rows_v, ones_v, zvec_v,
         sem_g, sem_s, sem_d, sem_i):
  c = lax.axis_index("c")
  s = lax.axis_index("s")
  gw = c * NS + s
  epw = E // NW
  nb = 78
  e_lo = gw * epw

  pltpu.async_copy(srcs.at[pl.ds(e_lo, epw)], src_v, sem_i)

  _zero_2d(rows_v.at[0], EB)
  _fill_1d(ones_v, EB, 1.0)
  _fill_1d(zvec_v, 1024, 0.0)

  def sidx(j):
    return src_v.at[pl.ds(j * EB, EB)]

  for j in range(6):
    pltpu.async_copy(rows_v.at[0].at[pl.ds(0, 104)],
                     acc.at[pl.ds(s * 624 + j * 104, 104)], sem_s)
  @pl.when(s == 0)
  def _():
    pltpu.sync_copy(rows_v.at[0].at[pl.ds(0, 16)], acc.at[pl.ds(NS * 624, 16)])
  @pl.when(s < 10)
  def _():
    pltpu.sync_copy(zvec_v.at[pl.ds(0, 1000)], degacc.at[pl.ds(s * 1000, 1000)])
  for j in range(6):
    pltpu.make_async_copy(rows_v.at[0].at[pl.ds(0, 104)],
                          acc.at[pl.ds(s * 624 + j * 104, 104)], sem_s).wait()
  pltpu.make_async_copy(srcs.at[pl.ds(e_lo, epw)], src_v, sem_i).wait()
  plsc.subcore_barrier()

  def idx_start(j, k):
    pltpu.async_copy(dsts.at[pl.ds(e_lo + j * EB, EB)], dst_b[k], sem_i)

  def idx_wait(k):
    pltpu.make_async_copy(dsts.at[pl.ds(0, EB)], dst_b[k], sem_i).wait()

  idx_start(0, 0)
  idx_start(1, 1)
  pltpu.async_copy(z1.at[sidx(0)], rows_v.at[0], sem_g)

  def outer(J, _):
    for k in range(4):
      j = J * 4 + k
      r = k % 2
      r1 = (k + 1) % 2
      kp = (k + 3) % 4
      k2 = (k + 2) % 4
      @pl.when(jnp.logical_and(j >= 1, j < nb + 1))
      def _():
        pltpu.make_async_copy(rows_v.at[r1], acc.at[dst_b[kp]], sem_s).wait()
      @pl.when(j + 1 < nb)
      def _():
        pltpu.async_copy(z1.at[sidx(j + 1)], rows_v.at[r1], sem_g)
      @pl.when(j < nb)
      def _():
        idx_wait(k)
        pltpu.make_async_copy(z1.at[sidx(j)], rows_v.at[r], sem_g).wait()
        pltpu.make_async_copy(rows_v.at[r], acc.at[dst_b[k]], sem_s
                              ).start(add=True)
        pltpu.make_async_copy(ones_v, degacc.at[dst_b[k]], sem_d
                              ).start(add=True)
      @pl.when(jnp.logical_and(j >= 1, j < nb + 1))
      def _():
        pltpu.make_async_copy(ones_v, degacc.at[dst_b[kp]], sem_d).wait()
      @pl.when(j + 2 < nb)
      def _():
        idx_start(j + 2, k2)
    return 0
  lax.fori_loop(0, (78 + 1 + 3) // 4 + 1, outer, 0)

  et = nb * EB
  rt = rows_v.at[0].at[pl.ds(0, 16)]
  pltpu.sync_copy(dsts.at[pl.ds(e_lo + et, 16)], dst_t)
  pltpu.async_copy(z1.at[src_v.at[pl.ds(et, 16)]], rt, sem_g).wait()
  pltpu.sync_copy(rt, acc.at[dst_t], add=True)
  pltpu.sync_copy(ones_v.at[pl.ds(0, 16)], degacc.at[dst_t], add=True)

  plsc.subcore_barrier()
  pltpu.sync_copy(acc.at[pl.ds(s * 624, 624)],
                  aggz_out.at[c, pl.ds(s * 624, 624)])
  @pl.when(s == 0)
  def _():
    pltpu.sync_copy(acc.at[pl.ds(NS * 624, 16)],
                    aggz_out.at[c, pl.ds(NS * 624, 16)])
  @pl.when(s < 10)
  def _():
    pltpu.sync_copy(degacc.at[pl.ds(s * 1000, 1000)], zvec_v.at[pl.ds(0, 1000)])
  @pl.when(jnp.logical_and(c == 0, s < 10))
  def _():
    pltpu.sync_copy(zvec_v.at[pl.ds(0, 1000)], deg_out0.at[pl.ds(s * 1000, 1000)])
  @pl.when(jnp.logical_and(c == 1, s < 10))
  def _():
    pltpu.sync_copy(zvec_v.at[pl.ds(0, 1000)], deg_out1.at[pl.ds(s * 1000, 1000)])


@functools.partial(
    pl.kernel,
    out_type=(
        jax.ShapeDtypeStruct((N,), jnp.float32),
        jax.ShapeDtypeStruct((N,), jnp.float32),
    ),
    mesh=_mesh,
    scratch_types=(
        pltpu.VMEM_SHARED((N,), jnp.float32),
        pltpu.VMEM((10240,), jnp.int32),
        pltpu.VMEM((10240,), jnp.int32),
        pltpu.VMEM((10240,), jnp.float32),
        pltpu.VMEM((2560,), jnp.int32),
        pltpu.VMEM((2560,), jnp.int32),
        pltpu.VMEM((2560,), jnp.float32),
        pltpu.VMEM((1024,), jnp.float32),
        pltpu.SemaphoreType.DMA,
    ),
)
def _sc2(n2, srcs, dsts, agg_out0, agg_out1, acc, src_v, dst_v, vals_v,
         src_s, dst_s, vals_s, zvec_v, sem):
  c = lax.axis_index("c")
  s = lax.axis_index("s")
  gw = c * NS + s

  _fill_1d(zvec_v, 1024, 0.0)
  @pl.when(s < 10)
  def _():
    pltpu.sync_copy(zvec_v.at[pl.ds(0, 1000)], acc.at[pl.ds(s * 1000, 1000)])
  plsc.subcore_barrier()

  @pl.when(gw < 31)
  def _():
    e0 = gw * 10240
    pltpu.sync_copy(srcs.at[pl.ds(e0, 10240)], src_v)
    pltpu.sync_copy(dsts.at[pl.ds(e0, 10240)], dst_v)
    pltpu.async_copy(n2.at[src_v], vals_v, sem).wait()
    pltpu.sync_copy(vals_v, acc.at[dst_v], add=True)
  @pl.when(gw == 31)
  def _():
    pltpu.sync_copy(srcs.at[pl.ds(317440, 2560)], src_s)
    pltpu.sync_copy(dsts.at[pl.ds(317440, 2560)], dst_s)
    pltpu.async_copy(n2.at[src_s], vals_s, sem).wait()
    pltpu.sync_copy(vals_s, acc.at[dst_s], add=True)

  plsc.subcore_barrier()
  @pl.when(s < 10)
  def _():
    pltpu.sync_copy(acc.at[pl.ds(s * 1000, 1000)], zvec_v.at[pl.ds(0, 1000)])
  @pl.when(jnp.logical_and(c == 0, s < 10))
  def _():
    pltpu.sync_copy(zvec_v.at[pl.ds(0, 1000)], agg_out0.at[pl.ds(s * 1000, 1000)])
  @pl.when(jnp.logical_and(c == 1, s < 10))
  def _():
    pltpu.sync_copy(zvec_v.at[pl.ds(0, 1000)], agg_out1.at[pl.ds(s * 1000, 1000)])


def _tc1_body(x_ref, wn_ref, ws_ref, z1_ref, s1_ref):
  xb = x_ref[...]
  z1_ref[...] = jnp.dot(xb, wn_ref[...], preferred_element_type=jnp.float32,
                        precision=HI)
  s1_ref[...] = jnp.dot(xb, ws_ref[...], preferred_element_type=jnp.float32,
                        precision=HI)


def _tc2_body(s1_ref, aggz_ref, d0_ref, d1_ref, bs1_ref, bn1_ref,
              ws2_ref, wn2_ref, n2_ref, s2_ref):
  deg = jnp.maximum(d0_ref[...] + d1_ref[...], 1.0)
  inv = 1.0 / deg
  agg = (aggz_ref[0] + aggz_ref[1]) * inv
  h = jnp.maximum(s1_ref[...] + bs1_ref[...] + bn1_ref[...] + agg, 0.0)
  s2_ref[...] = jnp.dot(h, ws2_ref[...], preferred_element_type=jnp.float32,
                        precision=HI)
  n2_ref[...] = jnp.dot(h, wn2_ref[...], preferred_element_type=jnp.float32,
                        precision=HI)


def _tc3_body(s2_ref, q0_ref, q1_ref, d0_ref, d1_ref, bs2_ref, bn2_ref,
              out_ref):
  deg = jnp.maximum(d0_ref[...] + d1_ref[...], 1.0)
  agg2 = q0_ref[...] + q1_ref[...]
  out_ref[...] = s2_ref[...] + bs2_ref[...] + bn2_ref[...] + agg2 / deg


def _tc1(x, wn, ws):
  return pl.pallas_call(
      _tc1_body,
      grid=(N // RB,),
      in_specs=[
          pl.BlockSpec((RB, D), lambda i: (i, 0)),
          pl.BlockSpec((D, D), lambda i: (0, 0)),
          pl.BlockSpec((D, D), lambda i: (0, 0)),
      ],
      out_specs=[
          pl.BlockSpec((RB, D), lambda i: (i, 0)),
          pl.BlockSpec((RB, D), lambda i: (i, 0)),
      ],
      out_shape=[
          jax.ShapeDtypeStruct((N, D), jnp.float32),
          jax.ShapeDtypeStruct((N, D), jnp.float32),
      ],
  )(x, wn, ws)


def _tc2(s1, aggz_p, d0, d1, bs1, bn1, ws2, wn2):
  return pl.pallas_call(
      _tc2_body,
      grid=(N // RB,),
      in_specs=[
          pl.BlockSpec((RB, D), lambda i: (i, 0)),
          pl.BlockSpec((NC, RB, D), lambda i: (0, i, 0)),
          pl.BlockSpec((RB, 1), lambda i: (i, 0)),
          pl.BlockSpec((RB, 1), lambda i: (i, 0)),
          pl.BlockSpec((1, D), lambda i: (0, 0)),
          pl.BlockSpec((1, D), lambda i: (0, 0)),
          pl.BlockSpec((D, 1), lambda i: (0, 0)),
          pl.BlockSpec((D, 1), lambda i: (0, 0)),
      ],
      out_specs=[
          pl.BlockSpec((RB, 1), lambda i: (i, 0)),
          pl.BlockSpec((RB, 1), lambda i: (i, 0)),
      ],
      out_shape=[
          jax.ShapeDtypeStruct((N, 1), jnp.float32),
          jax.ShapeDtypeStruct((N, 1), jnp.float32),
      ],
  )(s1, aggz_p, d0, d1, bs1, bn1, ws2, wn2)


def _tc3(s2, q0, q1, d0, d1, bs2, bn2):
  return pl.pallas_call(
      _tc3_body,
      grid=(N // RB,),
      in_specs=[
          pl.BlockSpec((RB, 1), lambda i: (i, 0)),
          pl.BlockSpec((RB, 1), lambda i: (i, 0)),
          pl.BlockSpec((RB, 1), lambda i: (i, 0)),
          pl.BlockSpec((RB, 1), lambda i: (i, 0)),
          pl.BlockSpec((RB, 1), lambda i: (i, 0)),
          pl.BlockSpec((1, 1), lambda i: (0, 0)),
          pl.BlockSpec((1, 1), lambda i: (0, 0)),
      ],
      out_specs=pl.BlockSpec((RB, 1), lambda i: (i, 0)),
      out_shape=jax.ShapeDtypeStruct((N, 1), jnp.float32),
  )(s2, q0, q1, d0, d1, bs2, bn2)


def kernel(x, edge_index, W_self1, b_self1, W_neigh1, b_neigh1,
           W_self2, b_self2, W_neigh2, b_neigh2):
  src = edge_index[0].astype(jnp.int32)
  dst = edge_index[1].astype(jnp.int32)

  z1, s1 = _tc1(x, W_neigh1, W_self1)
  aggz_p, deg0, deg1 = _sc1(z1, src, dst)
  d0 = deg0.reshape(N, 1)
  d1 = deg1.reshape(N, 1)

  n2, s2 = _tc2(s1, aggz_p, d0, d1, b_self1.reshape(1, D),
                b_neigh1.reshape(1, D), W_self2, W_neigh2)

  agg20, agg21 = _sc2(n2.reshape(N), src, dst)
  return _tc3(s2, agg20.reshape(N, 1), agg21.reshape(N, 1), d0, d1,
              b_self2.reshape(1, 1), b_neigh2.reshape(1, 1))
